# baseline (speedup 1.0000x reference)
import functools

import jax
import jax.numpy as jnp
from jax import lax
from jax.experimental import pallas as pl
from jax.experimental.pallas import tpu as pltpu
from jax.experimental.pallas import tpu_sc as plsc

N_NODES = 10000
N_PAD = 10240
E = 320000
NW = 32
NS = 16
L = 16
CH = N_PAD // NS
UNROLL = 8

W0 = 13952
W1 = 6144
G0 = W0 // L
G1 = W1 // L
WSUM = W0 + W1
CORE1_BASE = NS * W0
OVL_G = (NS * WSUM - E) // L
PAD_NODE = 10016
WMAX = max(W0, W1)


def _pad_window(edges):
    pv = jnp.full((L,), PAD_NODE, jnp.int32)

    @plsc.parallel_loop(0, OVL_G, unroll=UNROLL)
    def pad_body(j):
        edges[0, pl.ds(j * L, L)] = pv
        edges[1, pl.ds(j * L, L)] = pv


def _newton_rsqrt(x):
    xi = lax.bitcast_convert_type(x, jnp.int32)
    yi = jnp.int32(0x5F3759DF) - lax.shift_right_logical(xi, 1)
    y = lax.bitcast_convert_type(yi, jnp.float32)
    hx = x * 0.5
    y = y * (1.5 - hx * y * y)
    y = y * (1.5 - hx * y * y)
    y = y * (1.5 - hx * y * y)
    return y


@functools.partial(
    pl.kernel,
    out_type=[jax.ShapeDtypeStruct((NW, N_PAD), jnp.float32),
              jax.ShapeDtypeStruct((NW, N_PAD), jnp.float32)],
    mesh=plsc.VectorSubcoreMesh(core_axis_name="c", subcore_axis_name="s"),
    compiler_params=pltpu.CompilerParams(needs_layout_passes=False),
    scratch_types=[
        pltpu.VMEM((2, WMAX), jnp.int32),
        pltpu.VMEM((2, WMAX), jnp.int32),
        pltpu.VMEM((N_PAD,), jnp.float32),
        pltpu.VMEM((N_PAD,), jnp.float32),
        pltpu.VMEM((N_PAD,), jnp.float32),
        pltpu.VMEM((NS, CH), jnp.float32),
        pltpu.VMEM((CH,), jnp.float32),
        pltpu.VMEM_SHARED((NS, N_PAD), jnp.float32),
        pltpu.VMEM_SHARED((N_PAD,), jnp.float32),
        pltpu.SemaphoreType.DMA,
    ],
)
def _sc_gcn(ei_hbm, deg_hbm, c_hbm, edges, edges2, hs, hd, nd, blk, ndc,
            shr, shrn, sem):
    cid = lax.axis_index("c")
    sid = lax.axis_index("s")
    wid = cid * NS + sid
    base0 = sid * W0
    base1 = jnp.minimum(CORE1_BASE + sid * W1, E - W1)
    last1 = sid == NS - 1

    zf = jnp.zeros((L,), jnp.float32)
    ones = jnp.ones((L,), jnp.float32)

    def _phase_a(w_own, w_sib, own_b, sib_b, pad_own, pad_sib):
        cp2 = pltpu.async_copy(
            ei_hbm.at[:, pl.ds(sib_b, w_sib)], edges2.at[:, :w_sib], sem)
        pltpu.sync_copy(ei_hbm.at[:, pl.ds(own_b, w_own)],
                        edges.at[:, :w_own])

        @pl.when(pad_own)
        def _():
            _pad_window(edges)

        @plsc.parallel_loop(0, N_PAD // L, unroll=UNROLL)
        def zero_body(j):
            hs[pl.ds(j * L, L)] = zf
            hd[pl.ds(j * L, L)] = zf

        @plsc.parallel_loop(0, w_own // L, unroll=UNROLL)
        def edge_body_a(i):
            plsc.addupdate_scatter(hs, [edges[0, pl.ds(i * L, L)]], ones)
            plsc.addupdate_scatter(hd, [edges[1, pl.ds(i * L, L)]], ones)

        cp2.wait()

        @pl.when(pad_sib)
        def _():
            _pad_window(edges2)

        @plsc.parallel_loop(0, w_sib // L, unroll=UNROLL)
        def edge_body_b(i):
            plsc.addupdate_scatter(hd, [edges2[1, pl.ds(i * L, L)]], ones)

    never = sid < 0
    @pl.when(cid == 0)
    def _():
        _phase_a(W0, W1, base0, base1, never, last1)

    @pl.when(cid == 1)
    def _():
        _phase_a(W1, W0, base1, base0, last1, never)

    pltpu.sync_copy(hd, shr.at[sid])
    pltpu.sync_copy(hs, deg_hbm.at[wid])
    plsc.subcore_barrier()
    col0 = sid * CH
    pltpu.sync_copy(shr.at[:, pl.ds(col0, CH)], blk)

    @plsc.parallel_loop(0, CH // L, unroll=2)
    def red_body(k):
        d = blk[0, pl.ds(k * L, L)]
        for r in range(1, NS):
            d = d + blk[r, pl.ds(k * L, L)]
        ndc[pl.ds(k * L, L)] = _newton_rsqrt(jnp.maximum(d, 1.0))

    pltpu.sync_copy(ndc, shrn.at[pl.ds(col0, CH)])

    @plsc.parallel_loop(0, N_PAD // L, unroll=UNROLL)
    def zero_body2(j):
        hs[pl.ds(j * L, L)] = zf

    plsc.subcore_barrier()
    pltpu.sync_copy(shrn, nd)

    @pl.when(cid == 0)
    def _():
        @plsc.parallel_loop(0, G0, unroll=UNROLL)
        def edge_body2(i):
            v = plsc.load_gather(nd, [edges[1, pl.ds(i * L, L)]])
            plsc.addupdate_scatter(hs, [edges[0, pl.ds(i * L, L)]], v)

    @pl.when(cid == 1)
    def _():
        @plsc.parallel_loop(0, G1, unroll=UNROLL)
        def edge_body2(i):
            v = plsc.load_gather(nd, [edges[1, pl.ds(i * L, L)]])
            plsc.addupdate_scatter(hs, [edges[0, pl.ds(i * L, L)]], v)

    pltpu.sync_copy(hs, c_hbm.at[wid])


def _final_body(degp_ref, cp_ref, f_ref, w1_ref, b1_ref, w2_ref, b2_ref,
                w3_ref, b3_ref, out_ref):
    dsum = jnp.sum(degp_ref[...], axis=0)[None, :]
    ns = lax.rsqrt(jnp.maximum(dsum, 1.0))
    c = jnp.sum(cp_ref[...], axis=0)[None, :] * ns
    cs = c[:, :N_NODES]
    pooled = jnp.dot(cs, f_ref[...], preferred_element_type=jnp.float32)
    pooled = pooled * (1.0 / N_NODES)
    g = jnp.dot(pooled, w1_ref[...], preferred_element_type=jnp.float32) + b1_ref[...][None, :]
    h = jnp.dot(g, w2_ref[...], preferred_element_type=jnp.float32) + b2_ref[...][None, :]
    h = jnp.maximum(h, 0.0)
    o = jnp.dot(h, w3_ref[...], preferred_element_type=jnp.float32) + b3_ref[...][None, :]
    out_ref[...] = jax.nn.sigmoid(o)


def kernel(features, edge_index, W1, b1, W2, b2, W3, b3):
    ei = edge_index.astype(jnp.int32)
    deg_parts, c_parts = _sc_gcn(ei)
    out = pl.pallas_call(
        _final_body,
        out_shape=jax.ShapeDtypeStruct((1, 1), jnp.float32),
    )(deg_parts, c_parts, features, W1, b1, W2, b2, W3, b3)
    return out


# speedup vs baseline: 1.0041x; 1.0041x over previous
import functools

import jax
import jax.numpy as jnp
from jax import lax
from jax.experimental import pallas as pl
from jax.experimental.pallas import tpu as pltpu
from jax.experimental.pallas import tpu_sc as plsc

N_NODES = 10000
N_PAD = 10240
E = 320000
NW = 32
NS = 16
L = 16
CH = N_PAD // NS
UNROLL = 8

W0 = 12928
W1 = 7168
G0 = W0 // L
G1 = W1 // L
WSUM = W0 + W1
CORE1_BASE = NS * W0
OVL_G = (NS * WSUM - E) // L
PAD_NODE = 10016
WMAX = max(W0, W1)


def _pad_window(edges):
    pv = jnp.full((L,), PAD_NODE, jnp.int32)

    @plsc.parallel_loop(0, OVL_G, unroll=UNROLL)
    def pad_body(j):
        edges[0, pl.ds(j * L, L)] = pv
        edges[1, pl.ds(j * L, L)] = pv


def _newton_rsqrt(x):
    xi = lax.bitcast_convert_type(x, jnp.int32)
    yi = jnp.int32(0x5F3759DF) - lax.shift_right_logical(xi, 1)
    y = lax.bitcast_convert_type(yi, jnp.float32)
    hx = x * 0.5
    y = y * (1.5 - hx * y * y)
    y = y * (1.5 - hx * y * y)
    y = y * (1.5 - hx * y * y)
    return y


@functools.partial(
    pl.kernel,
    out_type=[jax.ShapeDtypeStruct((NW, N_PAD), jnp.float32),
              jax.ShapeDtypeStruct((NW, N_PAD), jnp.float32)],
    mesh=plsc.VectorSubcoreMesh(core_axis_name="c", subcore_axis_name="s"),
    compiler_params=pltpu.CompilerParams(needs_layout_passes=False),
    scratch_types=[
        pltpu.VMEM((2, WMAX), jnp.int32),
        pltpu.VMEM((2, WMAX), jnp.int32),
        pltpu.VMEM((N_PAD,), jnp.float32),
        pltpu.VMEM((N_PAD,), jnp.float32),
        pltpu.VMEM((N_PAD,), jnp.float32),
        pltpu.VMEM((NS, CH), jnp.float32),
        pltpu.VMEM((CH,), jnp.float32),
        pltpu.VMEM_SHARED((NS, N_PAD), jnp.float32),
        pltpu.VMEM_SHARED((N_PAD,), jnp.float32),
        pltpu.SemaphoreType.DMA,
    ],
)
def _sc_gcn(ei_hbm, deg_hbm, c_hbm, edges, edges2, hs, hd, nd, blk, ndc,
            shr, shrn, sem):
    cid = lax.axis_index("c")
    sid = lax.axis_index("s")
    wid = cid * NS + sid
    base0 = sid * W0
    base1 = jnp.minimum(CORE1_BASE + sid * W1, E - W1)
    last1 = sid == NS - 1

    zf = jnp.zeros((L,), jnp.float32)
    ones = jnp.ones((L,), jnp.float32)

    def _phase_a(w_own, w_sib, own_b, sib_b, pad_own, pad_sib):
        cp2 = pltpu.async_copy(
            ei_hbm.at[:, pl.ds(sib_b, w_sib)], edges2.at[:, :w_sib], sem)
        pltpu.sync_copy(ei_hbm.at[:, pl.ds(own_b, w_own)],
                        edges.at[:, :w_own])

        @pl.when(pad_own)
        def _():
            _pad_window(edges)

        @plsc.parallel_loop(0, N_PAD // L, unroll=UNROLL)
        def zero_body(j):
            hs[pl.ds(j * L, L)] = zf
            hd[pl.ds(j * L, L)] = zf

        @plsc.parallel_loop(0, w_own // L, unroll=UNROLL)
        def edge_body_a(i):
            plsc.addupdate_scatter(hs, [edges[0, pl.ds(i * L, L)]], ones)
            plsc.addupdate_scatter(hd, [edges[1, pl.ds(i * L, L)]], ones)

        cp2.wait()

        @pl.when(pad_sib)
        def _():
            _pad_window(edges2)

        @plsc.parallel_loop(0, w_sib // L, unroll=UNROLL)
        def edge_body_b(i):
            plsc.addupdate_scatter(hd, [edges2[1, pl.ds(i * L, L)]], ones)

    never = sid < 0
    @pl.when(cid == 0)
    def _():
        _phase_a(W0, W1, base0, base1, never, last1)

    @pl.when(cid == 1)
    def _():
        _phase_a(W1, W0, base1, base0, last1, never)

    pltpu.sync_copy(hd, shr.at[sid])
    pltpu.sync_copy(hs, deg_hbm.at[wid])
    plsc.subcore_barrier()
    col0 = sid * CH
    pltpu.sync_copy(shr.at[:, pl.ds(col0, CH)], blk)

    @plsc.parallel_loop(0, CH // L, unroll=2)
    def red_body(k):
        d = blk[0, pl.ds(k * L, L)]
        for r in range(1, NS):
            d = d + blk[r, pl.ds(k * L, L)]
        ndc[pl.ds(k * L, L)] = _newton_rsqrt(jnp.maximum(d, 1.0))

    pltpu.sync_copy(ndc, shrn.at[pl.ds(col0, CH)])

    @plsc.parallel_loop(0, N_PAD // L, unroll=UNROLL)
    def zero_body2(j):
        hs[pl.ds(j * L, L)] = zf

    plsc.subcore_barrier()
    pltpu.sync_copy(shrn, nd)

    @pl.when(cid == 0)
    def _():
        @plsc.parallel_loop(0, G0, unroll=UNROLL)
        def edge_body2(i):
            v = plsc.load_gather(nd, [edges[1, pl.ds(i * L, L)]])
            plsc.addupdate_scatter(hs, [edges[0, pl.ds(i * L, L)]], v)

    @pl.when(cid == 1)
    def _():
        @plsc.parallel_loop(0, G1, unroll=UNROLL)
        def edge_body2(i):
            v = plsc.load_gather(nd, [edges[1, pl.ds(i * L, L)]])
            plsc.addupdate_scatter(hs, [edges[0, pl.ds(i * L, L)]], v)

    pltpu.sync_copy(hs, c_hbm.at[wid])


def _final_body(degp_ref, cp_ref, f_ref, w1_ref, b1_ref, w2_ref, b2_ref,
                w3_ref, b3_ref, out_ref):
    dsum = jnp.sum(degp_ref[...], axis=0)[None, :]
    ns = lax.rsqrt(jnp.maximum(dsum, 1.0))
    c = jnp.sum(cp_ref[...], axis=0)[None, :] * ns
    cs = c[:, :N_NODES]
    pooled = jnp.dot(cs, f_ref[...], preferred_element_type=jnp.float32)
    pooled = pooled * (1.0 / N_NODES)
    g = jnp.dot(pooled, w1_ref[...], preferred_element_type=jnp.float32) + b1_ref[...][None, :]
    h = jnp.dot(g, w2_ref[...], preferred_element_type=jnp.float32) + b2_ref[...][None, :]
    h = jnp.maximum(h, 0.0)
    o = jnp.dot(h, w3_ref[...], preferred_element_type=jnp.float32) + b3_ref[...][None, :]
    out_ref[...] = jax.nn.sigmoid(o)


def kernel(features, edge_index, W1, b1, W2, b2, W3, b3):
    ei = edge_index.astype(jnp.int32)
    deg_parts, c_parts = _sc_gcn(ei)
    out = pl.pallas_call(
        _final_body,
        out_shape=jax.ShapeDtypeStruct((1, 1), jnp.float32),
    )(deg_parts, c_parts, features, W1, b1, W2, b2, W3, b3)
    return out
